# baseline (speedup 1.0000x reference)
import functools

import jax
import jax.numpy as jnp
from jax import lax
from jax.experimental import pallas as pl
from jax.experimental.pallas import tpu as pltpu
from jax.experimental.pallas import tpu_sc as plsc

N_EDGES = 320000
D_FEAT = 128
LANES = 16
VREGS_PER_ROW = D_FEAT // LANES

_info = plsc.get_sparse_core_info()
NC = _info.num_cores
NS = _info.num_subcores
NW = NC * NS
ROWS_PER_W = N_EDGES // NW
CHUNK = 80
N_CHUNKS = ROWS_PER_W // CHUNK
NSETS = 3


def _sc_body(h_hbm, idx_hbm, out_hbm,
             i0, i1, i2, r0, r1, r2, s0, s1, s2, o0, o1, o2,
             is0, is1, is2, g0, g1, g2, q0, q1, q2, w0, w1, w2):
    wid = lax.axis_index("s") * NC + lax.axis_index("c")
    base_w = wid * ROWS_PER_W
    idxb = (i0, i1, i2)
    rows = (r0, r1, r2)
    seq = (s0, s1, s2)
    out = (o0, o1, o2)
    isem = (is0, is1, is2)
    gsem = (g0, g1, g2)
    qsem = (q0, q1, q2)
    wsem = (w0, w1, w2)

    def start_idx(ci, b):
        pltpu.async_copy(
            idx_hbm.at[pl.ds(base_w + ci * CHUNK, CHUNK)], idxb[b], isem[b])

    def wait_idx(b):
        pltpu.make_async_copy(
            idx_hbm.at[pl.ds(0, CHUNK)], idxb[b], isem[b]).wait()

    HALF = CHUNK // 2

    def start_loads(ci, b):
        off = base_w + ci * CHUNK
        pltpu.async_copy(h_hbm.at[idxb[b].at[pl.ds(0, HALF)]],
                         rows[b].at[pl.ds(0, HALF)], gsem[b])
        pltpu.async_copy(h_hbm.at[idxb[b].at[pl.ds(HALF, HALF)]],
                         rows[b].at[pl.ds(HALF, HALF)], gsem[b])
        pltpu.async_copy(h_hbm.at[pl.ds(off, CHUNK)], seq[b], qsem[b])

    def wait_loads(b):
        pltpu.make_async_copy(h_hbm.at[idxb[b].at[pl.ds(0, HALF)]],
                              rows[b].at[pl.ds(0, HALF)], gsem[b]).wait()
        pltpu.make_async_copy(h_hbm.at[idxb[b].at[pl.ds(HALF, HALF)]],
                              rows[b].at[pl.ds(HALF, HALF)], gsem[b]).wait()
        pltpu.make_async_copy(h_hbm.at[pl.ds(0, CHUNK)], seq[b], qsem[b]).wait()

    def wait_wb(b):
        pltpu.make_async_copy(out[b], out_hbm.at[pl.ds(0, CHUNK)],
                              wsem[b]).wait()

    def slot(i, b):
        i = jnp.int32(i)
        pb = (b + 2) % NSETS

        @pl.when(i + 2 < N_CHUNKS)
        def _():
            wait_idx(pb)
            start_loads(i + 2, pb)

        wait_loads(b)

        @pl.when(i + 3 < N_CHUNKS)
        def _():
            start_idx(i + 3, b)

        @pl.when(i >= NSETS)
        def _():
            wait_wb(b)

        def row_body(j, _):
            for l in range(VREGS_PER_ROW):
                sl = pl.ds(l * LANES, LANES)
                out[b][j, sl] = (rows[b][j, sl] + seq[b][j, sl]) * 0.5
            return 0

        lax.fori_loop(0, CHUNK, row_body, 0)
        pltpu.async_copy(
            out[b], out_hbm.at[pl.ds(base_w + i * CHUNK, CHUNK)], wsem[b])

    pltpu.sync_copy(idx_hbm.at[pl.ds(base_w, CHUNK)], i0)
    pltpu.sync_copy(idx_hbm.at[pl.ds(base_w + CHUNK, CHUNK)], i1)
    start_idx(2, 2)
    start_loads(0, 0)
    start_loads(1, 1)

    def triple(k, _):
        slot(3 * k, 0)
        slot(3 * k + 1, 1)
        slot(3 * k + 2, 2)
        return 0

    n_full = N_CHUNKS // NSETS
    lax.fori_loop(0, n_full, triple, 0)
    slot(N_CHUNKS - 2, 0)
    slot(N_CHUNKS - 1, 1)
    wait_wb(2)
    wait_wb(0)
    wait_wb(1)


@jax.jit
def _message_sym(H, rev_index):
    mesh = plsc.VectorSubcoreMesh(core_axis_name="c", subcore_axis_name="s")
    fn = functools.partial(
        pl.kernel,
        mesh=mesh,
        out_type=jax.ShapeDtypeStruct((N_EDGES, D_FEAT), jnp.float32),
        scratch_types=(
            [pltpu.VMEM((CHUNK,), jnp.int32)] * 3
            + [pltpu.VMEM((CHUNK, D_FEAT), jnp.float32)] * 9
            + [pltpu.SemaphoreType.DMA] * 12
        ),
    )(_sc_body)
    return fn(H, rev_index)


def kernel(H, V, E, rev_index):
    return _message_sym(H, rev_index.astype(jnp.int32))


# speedup vs baseline: 1.0031x; 1.0031x over previous
import functools

import jax
import jax.numpy as jnp
from jax import lax
from jax.experimental import pallas as pl
from jax.experimental.pallas import tpu as pltpu
from jax.experimental.pallas import tpu_sc as plsc

N_EDGES = 320000
D_FEAT = 128
LANES = 16
VREGS_PER_ROW = D_FEAT // LANES

_info = plsc.get_sparse_core_info()
NC = _info.num_cores
NS = _info.num_subcores
NW = NC * NS
ROWS_PER_W = N_EDGES // NW
CHUNK = 80
N_CHUNKS = ROWS_PER_W // CHUNK
NSETS = 3


def _sc_body(h_hbm, idx_hbm, out_hbm,
             i0, i1, i2, r0, r1, r2, s0, s1, s2, o0, o1, o2,
             is0, is1, is2, g0, g1, g2, q0, q1, q2, w0, w1, w2):
    wid = lax.axis_index("s") * NC + lax.axis_index("c")
    base_w = wid * ROWS_PER_W
    idxb = (i0, i1, i2)
    rows = (r0, r1, r2)
    seq = (s0, s1, s2)
    out = (o0, o1, o2)
    isem = (is0, is1, is2)
    gsem = (g0, g1, g2)
    qsem = (q0, q1, q2)
    wsem = (w0, w1, w2)

    def start_idx(ci, b):
        pltpu.async_copy(
            idx_hbm.at[pl.ds(base_w + ci * CHUNK, CHUNK)], idxb[b], isem[b])

    def wait_idx(b):
        pltpu.make_async_copy(
            idx_hbm.at[pl.ds(0, CHUNK)], idxb[b], isem[b]).wait()

    def start_loads(ci, b):
        off = base_w + ci * CHUNK
        pltpu.async_copy(h_hbm.at[idxb[b]], rows[b], gsem[b])
        pltpu.async_copy(h_hbm.at[pl.ds(off, CHUNK)], seq[b], qsem[b])

    def wait_loads(b):
        pltpu.make_async_copy(h_hbm.at[idxb[b]], rows[b], gsem[b]).wait()
        pltpu.make_async_copy(h_hbm.at[pl.ds(0, CHUNK)], seq[b], qsem[b]).wait()

    def wait_wb(b):
        pltpu.make_async_copy(out[b], out_hbm.at[pl.ds(0, CHUNK)],
                              wsem[b]).wait()

    def slot(i, b):
        i = jnp.int32(i)
        pb = (b + 2) % NSETS

        @pl.when(i + 2 < N_CHUNKS)
        def _():
            wait_idx(pb)
            start_loads(i + 2, pb)

        wait_loads(b)

        @pl.when(i + 3 < N_CHUNKS)
        def _():
            start_idx(i + 3, b)

        @pl.when(i >= NSETS)
        def _():
            wait_wb(b)

        def row_body(j, _):
            for l in range(VREGS_PER_ROW):
                sl = pl.ds(l * LANES, LANES)
                out[b][j, sl] = (rows[b][j, sl] + seq[b][j, sl]) * 0.5
            return 0

        lax.fori_loop(0, CHUNK, row_body, 0)
        pltpu.async_copy(
            out[b], out_hbm.at[pl.ds(base_w + i * CHUNK, CHUNK)], wsem[b])

    pltpu.sync_copy(idx_hbm.at[pl.ds(base_w, CHUNK)], i0)
    pltpu.sync_copy(idx_hbm.at[pl.ds(base_w + CHUNK, CHUNK)], i1)
    start_idx(2, 2)
    start_loads(0, 0)
    start_loads(1, 1)

    def triple(k, _):
        slot(3 * k, 0)
        slot(3 * k + 1, 1)
        slot(3 * k + 2, 2)
        return 0

    n_full = N_CHUNKS // NSETS
    lax.fori_loop(0, n_full, triple, 0)
    slot(N_CHUNKS - 2, 0)
    slot(N_CHUNKS - 1, 1)
    wait_wb(2)
    wait_wb(0)
    wait_wb(1)


@jax.jit
def _message_sym(H, rev_index):
    mesh = plsc.VectorSubcoreMesh(core_axis_name="c", subcore_axis_name="s")
    fn = functools.partial(
        pl.kernel,
        mesh=mesh,
        out_type=jax.ShapeDtypeStruct((N_EDGES, D_FEAT), jnp.float32),
        scratch_types=(
            [pltpu.VMEM((CHUNK,), jnp.int32)] * 3
            + [pltpu.VMEM((CHUNK, D_FEAT), jnp.float32)] * 9
            + [pltpu.SemaphoreType.DMA] * 12
        ),
    )(_sc_body)
    return fn(H, rev_index)


def kernel(H, V, E, rev_index):
    return _message_sym(H, rev_index.astype(jnp.int32))
